# Initial kernel scaffold; baseline (speedup 1.0000x reference)
import jax
import jax.numpy as jnp
from jax.experimental import pallas as pl


def kernel(unary, binary, index1, index2):
    raise NotImplementedError("write your pallas kernel here")


# speedup vs baseline: 2.5935x; 2.5935x over previous
import functools

import jax
import jax.numpy as jnp
from jax import lax
from jax.experimental import pallas as pl
from jax.experimental.pallas import tpu as pltpu
from jax.experimental.pallas import tpu_sc as plsc


def kernel(unary, binary, index1, index2):
    N, D = unary.shape
    E, DB = binary.shape
    DO = 2 * D + DB

    info = plsc.get_sparse_core_info()
    NW = info.num_cores * info.num_subcores
    NC = info.num_cores

    assert E % NW == 0
    b_per_w = E // NW
    BC = 80
    assert b_per_w % BC == 0
    n_chunks = b_per_w // BC

    index1 = index1.astype(jnp.int32)
    index2 = index2.astype(jnp.int32)

    mesh = plsc.VectorSubcoreMesh(core_axis_name="c", subcore_axis_name="s")

    @functools.partial(
        pl.kernel,
        mesh=mesh,
        out_type=jax.ShapeDtypeStruct((E, DO), jnp.float32),
        scratch_types=[
            pltpu.VMEM((b_per_w,), jnp.int32),
            pltpu.VMEM((b_per_w,), jnp.int32),
            pltpu.VMEM((BC, D), jnp.float32),
            pltpu.VMEM((BC, D), jnp.float32),
            pltpu.VMEM((BC, DB), jnp.float32),
            pltpu.SemaphoreType.DMA,
            pltpu.SemaphoreType.DMA,
            pltpu.SemaphoreType.DMA,
        ],
    )
    def join(unary_hbm, binary_hbm, idx1_hbm, idx2_hbm, out_hbm,
             idx1_v, idx2_v, rows1_v, rows2_v, bin_v, sem1, sem2, sem3):
        wid = lax.axis_index("s") * NC + lax.axis_index("c")
        base = wid * b_per_w
        pltpu.sync_copy(idx1_hbm.at[pl.ds(base, b_per_w)], idx1_v)
        pltpu.sync_copy(idx2_hbm.at[pl.ds(base, b_per_w)], idx2_v)

        def chunk(i, carry):
            off = base + i * BC
            g1 = pltpu.async_copy(
                unary_hbm.at[idx1_v.at[pl.ds(i * BC, BC)]], rows1_v, sem1)
            g2 = pltpu.async_copy(
                unary_hbm.at[idx2_v.at[pl.ds(i * BC, BC)]], rows2_v, sem2)
            g3 = pltpu.async_copy(
                binary_hbm.at[pl.ds(off, BC)], bin_v, sem3)
            g1.wait()
            g2.wait()
            g3.wait()
            w1 = pltpu.async_copy(
                rows1_v, out_hbm.at[pl.ds(off, BC), pl.ds(0, D)], sem1)
            w2 = pltpu.async_copy(
                rows2_v, out_hbm.at[pl.ds(off, BC), pl.ds(D, D)], sem2)
            w3 = pltpu.async_copy(
                bin_v, out_hbm.at[pl.ds(off, BC), pl.ds(2 * D, DB)], sem3)
            w1.wait()
            w2.wait()
            w3.wait()
            return carry

        lax.fori_loop(0, n_chunks, chunk, 0)

    return join(unary, binary, index1, index2)


# speedup vs baseline: 2.8206x; 1.0876x over previous
import functools

import jax
import jax.numpy as jnp
from jax import lax
from jax.experimental import pallas as pl
from jax.experimental.pallas import tpu as pltpu
from jax.experimental.pallas import tpu_sc as plsc


def kernel(unary, binary, index1, index2):
    N, D = unary.shape
    E, DB = binary.shape
    DO = 2 * D + DB

    info = plsc.get_sparse_core_info()
    NW = info.num_cores * info.num_subcores
    NC = info.num_cores

    assert E % NW == 0
    b_per_w = E // NW
    BC = 80
    assert b_per_w % BC == 0
    n_chunks = b_per_w // BC
    NBUF = 3
    n_groups = (n_chunks + NBUF - 1) // NBUF
    assert n_chunks >= NBUF

    index1 = index1.astype(jnp.int32)
    index2 = index2.astype(jnp.int32)

    mesh = plsc.VectorSubcoreMesh(core_axis_name="c", subcore_axis_name="s")

    scratch = [
        pltpu.VMEM((b_per_w,), jnp.int32),
        pltpu.VMEM((b_per_w,), jnp.int32),
    ]
    for _ in range(NBUF):
        scratch += [
            pltpu.VMEM((BC, D), jnp.float32),
            pltpu.VMEM((BC, D), jnp.float32),
            pltpu.VMEM((BC, DB), jnp.float32),
        ]
    scratch += [pltpu.SemaphoreType.DMA] * (2 * NBUF)

    @functools.partial(
        pl.kernel,
        mesh=mesh,
        out_type=jax.ShapeDtypeStruct((E, DO), jnp.float32),
        scratch_types=scratch,
    )
    def join(unary_hbm, binary_hbm, idx1_hbm, idx2_hbm, out_hbm, *sc):
        idx1_v, idx2_v = sc[0], sc[1]
        bufs = [tuple(sc[2 + 3 * b:5 + 3 * b]) for b in range(NBUF)]
        gsem = sc[2 + 3 * NBUF:2 + 4 * NBUF]
        wsem = sc[2 + 4 * NBUF:2 + 5 * NBUF]

        wid = lax.axis_index("s") * NC + lax.axis_index("c")
        base = wid * b_per_w
        pltpu.sync_copy(idx1_hbm.at[pl.ds(base, b_per_w)], idx1_v)
        pltpu.sync_copy(idx2_hbm.at[pl.ds(base, b_per_w)], idx2_v)

        def issue_gathers(i, b):
            r1, r2, bb = bufs[b]
            off = base + i * BC
            pltpu.async_copy(
                unary_hbm.at[idx1_v.at[pl.ds(i * BC, BC)]], r1, gsem[b])
            pltpu.async_copy(
                unary_hbm.at[idx2_v.at[pl.ds(i * BC, BC)]], r2, gsem[b])
            pltpu.async_copy(binary_hbm.at[pl.ds(off, BC)], bb, gsem[b])

        def wait_gathers(b):
            r1, r2, bb = bufs[b]
            pltpu.make_async_copy(unary_hbm.at[pl.ds(0, BC)], r1, gsem[b]).wait()
            pltpu.make_async_copy(unary_hbm.at[pl.ds(0, BC)], r2, gsem[b]).wait()
            pltpu.make_async_copy(binary_hbm.at[pl.ds(0, BC)], bb, gsem[b]).wait()

        def issue_writes(i, b):
            r1, r2, bb = bufs[b]
            off = base + i * BC
            pltpu.async_copy(
                r1, out_hbm.at[pl.ds(off, BC), pl.ds(0, D)], wsem[b])
            pltpu.async_copy(
                r2, out_hbm.at[pl.ds(off, BC), pl.ds(D, D)], wsem[b])
            pltpu.async_copy(
                bb, out_hbm.at[pl.ds(off, BC), pl.ds(2 * D, DB)], wsem[b])

        def wait_writes(b):
            r1, r2, bb = bufs[b]
            pltpu.make_async_copy(
                r1, out_hbm.at[pl.ds(0, BC), pl.ds(0, D)], wsem[b]).wait()
            pltpu.make_async_copy(
                r2, out_hbm.at[pl.ds(0, BC), pl.ds(D, D)], wsem[b]).wait()
            pltpu.make_async_copy(
                bb, out_hbm.at[pl.ds(0, BC), pl.ds(2 * D, DB)], wsem[b]).wait()

        for b in range(NBUF):
            issue_gathers(b, b)

        def group(g, carry):
            for b in range(NBUF):
                i = g * NBUF + b

                @pl.when(i < n_chunks)
                def _():
                    wait_gathers(b)
                    issue_writes(i, b)

                @pl.when(i + NBUF < n_chunks)
                def _():
                    wait_writes(b)
                    issue_gathers(i + NBUF, b)

            return carry

        lax.fori_loop(0, n_groups, group, 0)

        for b in range(NBUF):
            wait_writes(b)

    return join(unary, binary, index1, index2)
